# Initial kernel scaffold; baseline (speedup 1.0000x reference)
import jax
import jax.numpy as jnp
from jax.experimental import pallas as pl


def kernel(pred_logits, pred_boxes, target_sizes, pred_names, mask_infos):
    raise NotImplementedError("write your pallas kernel here")


# speedup vs baseline: 5.6440x; 5.6440x over previous
import jax
import jax.numpy as jnp
from jax.experimental import pallas as pl
from jax.experimental.pallas import tpu as pltpu

_BS = 4
_Q = 300
_P = 64
_N = _BS * _P
_QP = 304
_TOPK = 100
_M = 128
_KEEP = 20
_KOUT = 24
_THR = 0.7


def _body(lg_ref, cx_ref, cy_ref, w_ref, h_ref, sc_ref, nm_ref, st_ref, en_ref,
          s_out, b1_out, b2_out, b3_out, b4_out, n_out, t_out, e_out,
          ss_ref, b1_ref, b2_ref, b3_ref, b4_ref, area_ref, supp_ref):
    rows = jax.lax.broadcasted_iota(jnp.int32, (_QP, _N), 0)

    x = lg_ref[...]
    prob = 0.5 * (jnp.tanh(0.5 * x) + 1.0)
    key0 = jnp.where(rows < _Q, prob, -1.0)

    sx = sc_ref[0:1, :]
    sy = sc_ref[1:2, :]
    cx = cx_ref[...]
    cy = cy_ref[...]
    bw = w_ref[...]
    bh = h_ref[...]
    x1 = (cx - 0.5 * bw) * sx
    y1 = (cy - 0.5 * bh) * sy
    x2 = (cx + 0.5 * bw) * sx
    y2 = (cy + 0.5 * bh) * sy

    zM = jnp.zeros((_M, _N), jnp.float32)
    ss_ref[...] = zM
    b1_ref[...] = zM
    b2_ref[...] = zM
    b3_ref[...] = zM
    b4_ref[...] = zM

    def topk_body(r, key):
        m = jnp.max(key, axis=0, keepdims=True)
        eq = key == m
        am = jnp.min(jnp.where(eq, rows, _QP), axis=0, keepdims=True)
        sel = rows == am
        ss_ref[pl.ds(r, 1), :] = m
        b1_ref[pl.ds(r, 1), :] = jnp.sum(jnp.where(sel, x1, 0.0), axis=0, keepdims=True)
        b2_ref[pl.ds(r, 1), :] = jnp.sum(jnp.where(sel, y1, 0.0), axis=0, keepdims=True)
        b3_ref[pl.ds(r, 1), :] = jnp.sum(jnp.where(sel, x2, 0.0), axis=0, keepdims=True)
        b4_ref[pl.ds(r, 1), :] = jnp.sum(jnp.where(sel, y2, 0.0), axis=0, keepdims=True)
        return jnp.where(sel, -2.0, key)

    jax.lax.fori_loop(0, _TOPK, topk_body, key0, unroll=4)

    rowsM = jax.lax.broadcasted_iota(jnp.int32, (_M, _N), 0)
    bb1 = b1_ref[...]
    bb2 = b2_ref[...]
    bb3 = b3_ref[...]
    bb4 = b4_ref[...]
    area = jnp.maximum(bb3 - bb1, 0.0) * jnp.maximum(bb4 - bb2, 0.0)
    area_ref[...] = area
    supp_ref[...] = jnp.where(rowsM < _TOPK, 0, 1)

    def nms_body(i, carry):
        xi1 = b1_ref[pl.ds(i, 1), :]
        xi2 = b2_ref[pl.ds(i, 1), :]
        xi3 = b3_ref[pl.ds(i, 1), :]
        xi4 = b4_ref[pl.ds(i, 1), :]
        ai = area_ref[pl.ds(i, 1), :]
        sup = supp_ref[...]
        act = supp_ref[pl.ds(i, 1), :] == 0
        iw = jnp.maximum(jnp.minimum(bb3, xi3) - jnp.maximum(bb1, xi1), 0.0)
        ih = jnp.maximum(jnp.minimum(bb4, xi4) - jnp.maximum(bb2, xi2), 0.0)
        inter = iw * ih
        union = area + ai - inter
        cond = act & (inter > _THR * jnp.maximum(union, 1e-9)) & (rowsM > i)
        supp_ref[...] = jnp.where(cond, 1, sup)
        return carry

    jax.lax.fori_loop(0, _TOPK, nms_body, 0, unroll=4)

    keep = supp_ref[...] == 0
    keepi = keep.astype(jnp.int32)
    c = keepi
    for sh in (1, 2, 4, 8, 16, 32, 64):
        c = c + jnp.pad(c, ((sh, 0), (0, 0)))[:_M]
    slot = c - keepi
    total = c[_M - 1:_M, :]
    ssv = ss_ref[...]
    names = nm_ref[0:1, :]
    starts = st_ref[0:1, :]
    ends = en_ref[0:1, :]
    for j in range(_KEEP):
        oh = keep & (slot == j)
        s_out[j:j + 1, :] = jnp.sum(jnp.where(oh, ssv, 0.0), axis=0, keepdims=True)
        b1_out[j:j + 1, :] = jnp.sum(jnp.where(oh, bb1, 0.0), axis=0, keepdims=True)
        b2_out[j:j + 1, :] = jnp.sum(jnp.where(oh, bb2, 0.0), axis=0, keepdims=True)
        b3_out[j:j + 1, :] = jnp.sum(jnp.where(oh, bb3, 0.0), axis=0, keepdims=True)
        b4_out[j:j + 1, :] = jnp.sum(jnp.where(oh, bb4, 0.0), axis=0, keepdims=True)
        valid_j = total > j
        s32 = jnp.int32
        n_out[j:j + 1, :] = jnp.where(valid_j, names, jnp.full((1, _N), -1, s32))
        t_out[j:j + 1, :] = jnp.where(valid_j, starts, jnp.full((1, _N), -1, s32))
        e_out[j:j + 1, :] = jnp.where(valid_j, ends, jnp.full((1, _N), -1, s32))
    for j in range(_KEEP, _KOUT):
        zr = jnp.zeros((1, _N), jnp.float32)
        ir = jnp.zeros((1, _N), jnp.int32)
        s_out[j:j + 1, :] = zr
        b1_out[j:j + 1, :] = zr
        b2_out[j:j + 1, :] = zr
        b3_out[j:j + 1, :] = zr
        b4_out[j:j + 1, :] = zr
        n_out[j:j + 1, :] = ir
        t_out[j:j + 1, :] = ir
        e_out[j:j + 1, :] = ir


def kernel(pred_logits, pred_boxes, target_sizes, pred_names, mask_infos):
    lg = pred_logits[:, 0, :, 1].reshape(_N, _Q).T
    lgT = jnp.pad(lg, ((0, _QP - _Q), (0, 0)))
    bx = jnp.transpose(pred_boxes[:, 0].reshape(_N, _Q, 4), (2, 1, 0))
    bxT = jnp.pad(bx, ((0, 0), (0, _QP - _Q), (0, 0)))
    img_w = target_sizes[:, 1]
    img_h = target_sizes[:, 0]
    sc = jnp.repeat(jnp.stack([img_w, img_h, img_w, img_h], 0), _P, axis=1)
    names = pred_names.reshape(1, _N).astype(jnp.int32)
    starts = mask_infos[..., 0].reshape(1, _N).astype(jnp.int32)
    ends = mask_infos[..., 1].reshape(1, _N).astype(jnp.int32)

    f32 = jnp.float32
    i32 = jnp.int32
    outs = pl.pallas_call(
        _body,
        out_shape=[
            jax.ShapeDtypeStruct((_KOUT, _N), f32),
            jax.ShapeDtypeStruct((_KOUT, _N), f32),
            jax.ShapeDtypeStruct((_KOUT, _N), f32),
            jax.ShapeDtypeStruct((_KOUT, _N), f32),
            jax.ShapeDtypeStruct((_KOUT, _N), f32),
            jax.ShapeDtypeStruct((_KOUT, _N), i32),
            jax.ShapeDtypeStruct((_KOUT, _N), i32),
            jax.ShapeDtypeStruct((_KOUT, _N), i32),
        ],
        scratch_shapes=[
            pltpu.VMEM((_M, _N), f32),
            pltpu.VMEM((_M, _N), f32),
            pltpu.VMEM((_M, _N), f32),
            pltpu.VMEM((_M, _N), f32),
            pltpu.VMEM((_M, _N), f32),
            pltpu.VMEM((_M, _N), f32),
            pltpu.VMEM((_M, _N), i32),
        ],
    )(lgT, bxT[0], bxT[1], bxT[2], bxT[3], sc, names, starts, ends)

    s_t, o1, o2, o3, o4, n_t, t_t, e_t = outs
    scores = s_t[:_KEEP].T.reshape(_BS, _P * _KEEP)
    boxes = jnp.stack([o1[:_KEEP], o2[:_KEEP], o3[:_KEEP], o4[:_KEEP]], axis=-1)
    boxes = jnp.transpose(boxes, (1, 0, 2)).reshape(_BS, _P * _KEEP, 4)
    names_o = n_t[:_KEEP].T.reshape(_BS, _P * _KEEP)
    starts_o = t_t[:_KEEP].T.reshape(_BS, _P * _KEEP)
    ends_o = e_t[:_KEEP].T.reshape(_BS, _P * _KEEP)
    return scores, boxes, names_o, starts_o, ends_o
